# baseline (speedup 1.0000x reference)
import functools

import jax
import jax.numpy as jnp
from jax import lax
from jax.experimental import pallas as pl
from jax.experimental.pallas import tpu as pltpu
from jax.experimental.pallas import tpu_sc as plsc

B, N, C = 32, 8192, 128
K = 16
L = 16
NC = 2
TB = 16
SB = B - TB
HN = N // 2
RB = 256
NCH = HN // RB
GRP = RB // L
CW = HN // L

NEG = float("-inf")
IBIG = 2**31 - 1


def _scores_body(x_ref, o_ref):
    o_ref[...] = jnp.max(x_ref[...], axis=2)


def _tc_scores_half(x):
    return pl.pallas_call(
        _scores_body,
        grid=(TB // 8, 8),
        in_specs=[pl.BlockSpec((8, 1024, 128), lambda i, j: (i, j, 0))],
        out_specs=pl.BlockSpec((8, 1024), lambda i, j: (i, j)),
        out_shape=jax.ShapeDtypeStruct((TB, N), jnp.float32),
    )(x)


def _select_top16(scores_v, cand_v, cand_i, m, base, topv_v, topi_v):
    lanes = lax.iota(jnp.int32, L)
    t0 = jnp.min(m)

    def p2(j, off):
        v = scores_v[pl.ds(j * L, L)]
        msk = v >= t0
        pos = off + plsc.cumsum(msk.astype(jnp.int32)) - 1
        plsc.store_scatter(cand_v, [pos], v, mask=msk)
        plsc.store_scatter(cand_i, [pos], base + j * L + lanes, mask=msk)
        return off + jnp.max(plsc.all_reduce_population_count(msk))

    c = lax.fori_loop(0, CW, p2, jnp.int32(0))

    pad_pos = jnp.full((L,), c, jnp.int32) + lanes
    plsc.store_scatter(cand_v, [pad_pos], jnp.full((L,), NEG, jnp.float32))
    plsc.store_scatter(cand_i, [pad_pos], jnp.full((L,), IBIG, jnp.int32))
    nch = (c + (L - 1)) // L
    lane0 = lanes == 0

    for s in range(K):
        def scan(j, carry):
            bv, bi, bp = carry
            v = cand_v[pl.ds(j * L, L)]
            ii = cand_i[pl.ds(j * L, L)]
            pp = lanes + j * L
            take = (v > bv) | ((v == bv) & (ii < bi))
            return (jnp.where(take, v, bv),
                    jnp.where(take, ii, bi),
                    jnp.where(take, pp, bp))

        bv, bi, bp = lax.fori_loop(
            0, nch, scan,
            (jnp.full((L,), NEG, jnp.float32),
             jnp.full((L,), IBIG, jnp.int32),
             jnp.full((L,), IBIG, jnp.int32)))
        mval = jnp.max(bv)
        eq = bv == mval
        mi = jnp.min(jnp.where(eq, bi, IBIG))
        pos = jnp.min(jnp.where(eq & (bi == mi), bp, IBIG))
        plsc.store_scatter(topv_v, [jnp.full((L,), s, jnp.int32)],
                           jnp.full((L,), mval, jnp.float32), mask=lane0)
        plsc.store_scatter(topi_v, [jnp.full((L,), s, jnp.int32)],
                           jnp.full((L,), mi, jnp.int32), mask=lane0)
        plsc.store_scatter(cand_v, [jnp.full((L,), pos, jnp.int32)],
                           jnp.full((L,), NEG, jnp.float32), mask=lane0)


def _merge_gather_mean(x2_hbm, out_row, mc_v, mc_i, idx_v, rows_v, out_v, sem):
    lanes = lax.iota(jnp.int32, L)
    lane0 = lanes == 0
    for s in range(K):
        v0, i0 = mc_v[pl.ds(0, L)], mc_i[pl.ds(0, L)]
        v1, i1 = mc_v[pl.ds(L, L)], mc_i[pl.ds(L, L)]
        take = (v1 > v0) | ((v1 == v0) & (i1 < i0))
        bv = jnp.where(take, v1, v0)
        bi = jnp.where(take, i1, i0)
        bp = jnp.where(take, lanes + L, lanes)
        mval = jnp.max(bv)
        eq = bv == mval
        mi = jnp.min(jnp.where(eq, bi, IBIG))
        pos = jnp.min(jnp.where(eq & (bi == mi), bp, IBIG))
        plsc.store_scatter(idx_v, [jnp.full((L,), s, jnp.int32)],
                           jnp.full((L,), mi, jnp.int32), mask=lane0)
        plsc.store_scatter(mc_v, [jnp.full((L,), pos, jnp.int32)],
                           jnp.full((L,), NEG, jnp.float32), mask=lane0)
    pltpu.async_copy(x2_hbm.at[idx_v], rows_v, sem).wait()
    for cc in range(C // L):
        acc = jnp.zeros((L,), jnp.float32)
        for r in range(K):
            acc = acc + rows_v[r, pl.ds(cc * L, L)]
        out_v[pl.ds(cc * L, L)] = acc * jnp.float32(1.0 / K)
    pltpu.sync_copy(out_v, out_row)


def _sca1_body(x2_hbm, cval_hbm, cidx_hbm,
               buf0, buf1, scores_v, cand_v, cand_i, topv_v, topi_v, m_v,
               dsem0, dsem1):
    cid = lax.axis_index("c")
    sid = lax.axis_index("s")
    bloc = sid // 2
    half = sid % 2
    b = TB + cid * 8 + bloc
    base_row = b * N + half * HN
    lanes = lax.iota(jnp.int32, L)

    m_v[...] = jnp.full((L,), NEG, jnp.float32)
    pltpu.make_async_copy(
        x2_hbm.at[pl.ds(base_row, RB)], buf0, dsem0).start()

    def chunk(o, buf, sem, nbuf, nsem):
        @pl.when(o < NCH - 1)
        def _():
            pltpu.make_async_copy(
                x2_hbm.at[pl.ds(base_row + (o + 1) * RB, RB)],
                nbuf, nsem).start()
        pltpu.make_async_copy(
            x2_hbm.at[pl.ds(base_row + o * RB, RB)], buf, sem).wait()

        def grp(g, _):
            rows = g * L + lanes
            acc = plsc.load_gather(buf, [rows, jnp.zeros((L,), jnp.int32)])
            for c in range(1, C):
                acc = jnp.maximum(
                    acc,
                    plsc.load_gather(buf, [rows, jnp.full((L,), c, jnp.int32)]))
            scores_v[pl.ds(o * RB + g * L, L)] = acc
            m_v[...] = jnp.maximum(m_v[...], acc)
            return 0

        lax.fori_loop(0, GRP, grp, 0)

    def step(o, _):
        @pl.when(o % 2 == 0)
        def _():
            chunk(o, buf0, dsem0, buf1, dsem1)

        @pl.when(o % 2 == 1)
        def _():
            chunk(o, buf1, dsem1, buf0, dsem0)
        return 0

    lax.fori_loop(0, NCH, step, 0)

    _select_top16(scores_v, cand_v, cand_i, m_v[...], base_row,
                  topv_v, topi_v)
    off = (cid * 8 + bloc) * (2 * K) + half * K
    pltpu.sync_copy(topv_v, cval_hbm.at[pl.ds(off, K)])
    pltpu.sync_copy(topi_v, cidx_hbm.at[pl.ds(off, K)])


def _sca2_body(cval_hbm, cidx_hbm, x2_hbm, out_hbm,
               mc_v, mc_i, idx_v, rows_v, out_v, gsem):
    cid = lax.axis_index("c")
    sid = lax.axis_index("s")
    bloc = sid // 2
    b2 = cid * 8 + bloc
    pltpu.sync_copy(cval_hbm.at[pl.ds(b2 * (2 * K), 2 * K)], mc_v)
    pltpu.sync_copy(cidx_hbm.at[pl.ds(b2 * (2 * K), 2 * K)], mc_i)
    _merge_gather_mean(x2_hbm, out_hbm.at[b2], mc_v, mc_i,
                       idx_v, rows_v, out_v, gsem)


def _scb_body(s1_hbm, x2_hbm, out_hbm,
              scores_v, cand_v, cand_i, topv_v, topi_v,
              mc_v, mc_i, idx_v, rows_v, out_v, gsem):
    cid = lax.axis_index("c")
    sid = lax.axis_index("s")
    bloc = sid // 2
    b = cid * 8 + bloc
    base_row = b * N

    def do_half(h):
        pltpu.sync_copy(s1_hbm.at[pl.ds(base_row + h * HN, HN)], scores_v)

        def p1(j, m):
            return jnp.maximum(m, scores_v[pl.ds(j * L, L)])

        m = lax.fori_loop(0, CW, p1, jnp.full((L,), NEG, jnp.float32))
        _select_top16(scores_v, cand_v, cand_i, m, base_row + h * HN,
                      topv_v, topi_v)
        mc_v[pl.ds(h * L, L)] = topv_v[...]
        mc_i[pl.ds(h * L, L)] = topi_v[...]

    do_half(0)
    do_half(1)
    _merge_gather_mean(x2_hbm, out_hbm.at[b], mc_v, mc_i,
                       idx_v, rows_v, out_v, gsem)


_SEL_SCRATCH = [
    pltpu.VMEM((HN,), jnp.float32),
    pltpu.VMEM((HN + L,), jnp.float32),
    pltpu.VMEM((HN + L,), jnp.int32),
    pltpu.VMEM((K,), jnp.float32),
    pltpu.VMEM((K,), jnp.int32),
]

_MERGE_SCRATCH = [
    pltpu.VMEM((2 * K,), jnp.float32),
    pltpu.VMEM((2 * K,), jnp.int32),
    pltpu.VMEM((K,), jnp.int32),
    pltpu.VMEM((K, C), jnp.float32),
    pltpu.VMEM((C,), jnp.float32),
    pltpu.SemaphoreType.DMA,
]

_MESH = dict(
    mesh=plsc.VectorSubcoreMesh(core_axis_name="c", subcore_axis_name="s"),
    compiler_params=pltpu.CompilerParams(needs_layout_passes=False),
)

_sc_a1 = functools.partial(
    pl.kernel,
    out_type=[jax.ShapeDtypeStruct((SB * 2 * K,), jnp.float32),
              jax.ShapeDtypeStruct((SB * 2 * K,), jnp.int32)],
    scratch_types=([
        pltpu.VMEM((RB, C), jnp.float32),
        pltpu.VMEM((RB, C), jnp.float32),
    ] + _SEL_SCRATCH + [
        pltpu.VMEM((L,), jnp.float32),
        pltpu.SemaphoreType.DMA,
        pltpu.SemaphoreType.DMA,
    ]),
    **_MESH,
)(_sca1_body)

_sc_a2 = functools.partial(
    pl.kernel,
    out_type=jax.ShapeDtypeStruct((SB, C), jnp.float32),
    scratch_types=list(_MERGE_SCRATCH),
    **_MESH,
)(_sca2_body)

_sc_b = functools.partial(
    pl.kernel,
    out_type=jax.ShapeDtypeStruct((TB, C), jnp.float32),
    scratch_types=(_SEL_SCRATCH + _MERGE_SCRATCH),
    **_MESH,
)(_scb_body)


@jax.jit
def kernel(input):
    x2 = input.reshape(B * N, C)
    cval, cidx = _sc_a1(x2)
    out_a = _sc_a2(cval, cidx, x2)
    scores = _tc_scores_half(input)
    out_b = _sc_b(scores.reshape(TB * N), x2)
    return jnp.concatenate([out_b, out_a], axis=0)[:, None, :]


# speedup vs baseline: 2.3266x; 2.3266x over previous
import functools

import jax
import jax.numpy as jnp
from jax import lax
from jax.experimental import pallas as pl
from jax.experimental.pallas import tpu as pltpu
from jax.experimental.pallas import tpu_sc as plsc

B, N, C = 32, 8192, 128
K = 16
L = 16
NC = 2
TB = 16
SB = B - TB
HN = N // 2
RB = 256
NCH = HN // RB
GRP = RB // L
CW = HN // L

NEG = float("-inf")
IBIG = 2**31 - 1


def _scores_body(x_ref, o_ref):
    o_ref[...] = jnp.max(x_ref[...], axis=2)


def _tc_scores_half(x):
    return pl.pallas_call(
        _scores_body,
        grid=(TB // 8, 8),
        in_specs=[pl.BlockSpec((8, 1024, 128), lambda i, j: (i, j, 0))],
        out_specs=pl.BlockSpec((8, 1024), lambda i, j: (i, j)),
        out_shape=jax.ShapeDtypeStruct((TB, N), jnp.float32),
    )(x)


def _select_top16(scores_v, cand_v, cand_i, m, base, topv_v, topi_v):
    lanes = lax.iota(jnp.int32, L)
    t0 = jnp.min(m)

    def p2(j, off):
        v = scores_v[pl.ds(j * L, L)]
        msk = v >= t0
        pos = off + plsc.cumsum(msk.astype(jnp.int32)) - 1
        plsc.store_scatter(cand_v, [pos], v, mask=msk)
        plsc.store_scatter(cand_i, [pos], base + j * L + lanes, mask=msk)
        return off + jnp.max(plsc.all_reduce_population_count(msk))

    c = lax.fori_loop(0, CW, p2, jnp.int32(0))

    pad_pos = jnp.full((L,), c, jnp.int32) + lanes
    plsc.store_scatter(cand_v, [pad_pos], jnp.full((L,), NEG, jnp.float32))
    plsc.store_scatter(cand_i, [pad_pos], jnp.full((L,), IBIG, jnp.int32))
    nch = (c + (L - 1)) // L
    lane0 = lanes == 0

    for s in range(K):
        def scan(j, carry):
            bv, bi, bp = carry
            v = cand_v[pl.ds(j * L, L)]
            ii = cand_i[pl.ds(j * L, L)]
            pp = lanes + j * L
            take = (v > bv) | ((v == bv) & (ii < bi))
            return (jnp.where(take, v, bv),
                    jnp.where(take, ii, bi),
                    jnp.where(take, pp, bp))

        bv, bi, bp = lax.fori_loop(
            0, nch, scan,
            (jnp.full((L,), NEG, jnp.float32),
             jnp.full((L,), IBIG, jnp.int32),
             jnp.full((L,), IBIG, jnp.int32)))
        mval = jnp.max(bv)
        eq = bv == mval
        mi = jnp.min(jnp.where(eq, bi, IBIG))
        pos = jnp.min(jnp.where(eq & (bi == mi), bp, IBIG))
        plsc.store_scatter(topv_v, [jnp.full((L,), s, jnp.int32)],
                           jnp.full((L,), mval, jnp.float32), mask=lane0)
        plsc.store_scatter(topi_v, [jnp.full((L,), s, jnp.int32)],
                           jnp.full((L,), mi, jnp.int32), mask=lane0)
        plsc.store_scatter(cand_v, [jnp.full((L,), pos, jnp.int32)],
                           jnp.full((L,), NEG, jnp.float32), mask=lane0)


def _merge_gather_mean(x2_hbm, out_row, mc_v, mc_i, idx_v, rows_v, out_v, sem):
    lanes = lax.iota(jnp.int32, L)
    lane0 = lanes == 0
    for s in range(K):
        v0, i0 = mc_v[pl.ds(0, L)], mc_i[pl.ds(0, L)]
        v1, i1 = mc_v[pl.ds(L, L)], mc_i[pl.ds(L, L)]
        take = (v1 > v0) | ((v1 == v0) & (i1 < i0))
        bv = jnp.where(take, v1, v0)
        bi = jnp.where(take, i1, i0)
        bp = jnp.where(take, lanes + L, lanes)
        mval = jnp.max(bv)
        eq = bv == mval
        mi = jnp.min(jnp.where(eq, bi, IBIG))
        pos = jnp.min(jnp.where(eq & (bi == mi), bp, IBIG))
        plsc.store_scatter(idx_v, [jnp.full((L,), s, jnp.int32)],
                           jnp.full((L,), mi, jnp.int32), mask=lane0)
        plsc.store_scatter(mc_v, [jnp.full((L,), pos, jnp.int32)],
                           jnp.full((L,), NEG, jnp.float32), mask=lane0)
    pltpu.async_copy(x2_hbm.at[idx_v], rows_v, sem).wait()
    for cc in range(C // L):
        acc = jnp.zeros((L,), jnp.float32)
        for r in range(K):
            acc = acc + rows_v[r, pl.ds(cc * L, L)]
        out_v[pl.ds(cc * L, L)] = acc * jnp.float32(1.0 / K)
    pltpu.sync_copy(out_v, out_row)


def _sca1_body(x2_hbm, cval_hbm, cidx_hbm,
               buf0, buf1, scores_v, cand_v, cand_i, topv_v, topi_v, m_v,
               dsem0, dsem1):
    cid = lax.axis_index("c")
    sid = lax.axis_index("s")
    bloc = sid // 2
    half = sid % 2
    b = TB + cid * 8 + bloc
    base_row = b * N + half * HN
    lanes = lax.iota(jnp.int32, L)

    pltpu.make_async_copy(
        x2_hbm.at[pl.ds(base_row, RB)], buf0, dsem0).start()

    def chunk(o, buf, sem, nbuf, nsem):
        @pl.when(o < NCH - 1)
        def _():
            pltpu.make_async_copy(
                x2_hbm.at[pl.ds(base_row + (o + 1) * RB, RB)],
                nbuf, nsem).start()
        pltpu.make_async_copy(
            x2_hbm.at[pl.ds(base_row + o * RB, RB)], buf, sem).wait()

        lane15 = lanes == (L - 1)

        def grp(g, _):
            for r in range(L):
                row = g * L + r
                v = buf[row, pl.ds(0, L)]
                for c in range(1, C // L):
                    v = jnp.maximum(v, buf[row, pl.ds(c * L, L)])
                cm = plsc.cummax(v)
                plsc.store_scatter(
                    scores_v,
                    [jnp.full((L,), o * RB + g * L + r, jnp.int32)],
                    cm, mask=lane15)
            return 0

        lax.fori_loop(0, GRP, grp, 0)

    def step(o, _):
        @pl.when(o % 2 == 0)
        def _():
            chunk(o, buf0, dsem0, buf1, dsem1)

        @pl.when(o % 2 == 1)
        def _():
            chunk(o, buf1, dsem1, buf0, dsem0)
        return 0

    lax.fori_loop(0, NCH, step, 0)

    def p1(j, m):
        return jnp.maximum(m, scores_v[pl.ds(j * L, L)])

    m = lax.fori_loop(0, CW, p1, jnp.full((L,), NEG, jnp.float32))
    _select_top16(scores_v, cand_v, cand_i, m, base_row,
                  topv_v, topi_v)
    off = (cid * 8 + bloc) * (2 * K) + half * K
    pltpu.sync_copy(topv_v, cval_hbm.at[pl.ds(off, K)])
    pltpu.sync_copy(topi_v, cidx_hbm.at[pl.ds(off, K)])


def _sca2_body(cval_hbm, cidx_hbm, x2_hbm, out_hbm,
               mc_v, mc_i, idx_v, rows_v, out_v, gsem):
    cid = lax.axis_index("c")
    sid = lax.axis_index("s")
    bloc = sid // 2
    b2 = cid * 8 + bloc
    pltpu.sync_copy(cval_hbm.at[pl.ds(b2 * (2 * K), 2 * K)], mc_v)
    pltpu.sync_copy(cidx_hbm.at[pl.ds(b2 * (2 * K), 2 * K)], mc_i)
    _merge_gather_mean(x2_hbm, out_hbm.at[b2], mc_v, mc_i,
                       idx_v, rows_v, out_v, gsem)


def _scb_body(s1_hbm, x2_hbm, out_hbm,
              scores_v, cand_v, cand_i, topv_v, topi_v,
              mc_v, mc_i, idx_v, rows_v, out_v, gsem):
    cid = lax.axis_index("c")
    sid = lax.axis_index("s")
    bloc = sid // 2
    b = cid * 8 + bloc
    base_row = b * N

    def do_half(h):
        pltpu.sync_copy(s1_hbm.at[pl.ds(base_row + h * HN, HN)], scores_v)

        def p1(j, m):
            return jnp.maximum(m, scores_v[pl.ds(j * L, L)])

        m = lax.fori_loop(0, CW, p1, jnp.full((L,), NEG, jnp.float32))
        _select_top16(scores_v, cand_v, cand_i, m, base_row + h * HN,
                      topv_v, topi_v)
        mc_v[pl.ds(h * L, L)] = topv_v[...]
        mc_i[pl.ds(h * L, L)] = topi_v[...]

    do_half(0)
    do_half(1)
    _merge_gather_mean(x2_hbm, out_hbm.at[b], mc_v, mc_i,
                       idx_v, rows_v, out_v, gsem)


_SEL_SCRATCH = [
    pltpu.VMEM((HN,), jnp.float32),
    pltpu.VMEM((HN + L,), jnp.float32),
    pltpu.VMEM((HN + L,), jnp.int32),
    pltpu.VMEM((K,), jnp.float32),
    pltpu.VMEM((K,), jnp.int32),
]

_MERGE_SCRATCH = [
    pltpu.VMEM((2 * K,), jnp.float32),
    pltpu.VMEM((2 * K,), jnp.int32),
    pltpu.VMEM((K,), jnp.int32),
    pltpu.VMEM((K, C), jnp.float32),
    pltpu.VMEM((C,), jnp.float32),
    pltpu.SemaphoreType.DMA,
]

_MESH = dict(
    mesh=plsc.VectorSubcoreMesh(core_axis_name="c", subcore_axis_name="s"),
    compiler_params=pltpu.CompilerParams(needs_layout_passes=False),
)

_sc_a1 = functools.partial(
    pl.kernel,
    out_type=[jax.ShapeDtypeStruct((SB * 2 * K,), jnp.float32),
              jax.ShapeDtypeStruct((SB * 2 * K,), jnp.int32)],
    scratch_types=([
        pltpu.VMEM((RB, C), jnp.float32),
        pltpu.VMEM((RB, C), jnp.float32),
    ] + _SEL_SCRATCH + [
        pltpu.VMEM((L,), jnp.float32),
        pltpu.SemaphoreType.DMA,
        pltpu.SemaphoreType.DMA,
    ]),
    **_MESH,
)(_sca1_body)

_sc_a2 = functools.partial(
    pl.kernel,
    out_type=jax.ShapeDtypeStruct((SB, C), jnp.float32),
    scratch_types=list(_MERGE_SCRATCH),
    **_MESH,
)(_sca2_body)

_sc_b = functools.partial(
    pl.kernel,
    out_type=jax.ShapeDtypeStruct((TB, C), jnp.float32),
    scratch_types=(_SEL_SCRATCH + _MERGE_SCRATCH),
    **_MESH,
)(_scb_body)


@jax.jit
def kernel(input):
    x2 = input.reshape(B * N, C)
    cval, cidx = _sc_a1(x2)
    out_a = _sc_a2(cval, cidx, x2)
    scores = _tc_scores_half(input)
    out_b = _sc_b(scores.reshape(TB * N), x2)
    return jnp.concatenate([out_b, out_a], axis=0)[:, None, :]


# speedup vs baseline: 2.8778x; 1.2369x over previous
import functools

import jax
import jax.numpy as jnp
from jax import lax
from jax.experimental import pallas as pl
from jax.experimental.pallas import tpu as pltpu
from jax.experimental.pallas import tpu_sc as plsc

B, N, C = 32, 8192, 128
K = 16
L = 16
NC = 2
TB = 24
SB = B - TB
WPB = 4
WROWS = N // WPB
HN = N // 2
RB = 256
NCH = WROWS // RB
GRP = RB // L
CW = HN // L

NEG = float("-inf")
IBIG = 2**31 - 1


def _scores_body(x_ref, o_ref):
    o_ref[...] = jnp.max(x_ref[...], axis=2)


def _tc_scores_half(x):
    return pl.pallas_call(
        _scores_body,
        grid=(TB // 8, 8),
        in_specs=[pl.BlockSpec((8, 1024, 128), lambda i, j: (i, j, 0))],
        out_specs=pl.BlockSpec((8, 1024), lambda i, j: (i, j)),
        out_shape=jax.ShapeDtypeStruct((TB, N), jnp.float32),
    )(x)


def _select_top16(scores_v, cand_v, cand_i, m, base, topv_v, topi_v, cw):
    lanes = lax.iota(jnp.int32, L)
    t0 = jnp.min(m)

    def p2(j, off):
        v = scores_v[pl.ds(j * L, L)]
        msk = v >= t0
        pos = off + plsc.cumsum(msk.astype(jnp.int32)) - 1
        plsc.store_scatter(cand_v, [pos], v, mask=msk)
        plsc.store_scatter(cand_i, [pos], base + j * L + lanes, mask=msk)
        return off + jnp.max(plsc.all_reduce_population_count(msk))

    c = lax.fori_loop(0, cw, p2, jnp.int32(0))

    pad_pos = jnp.full((L,), c, jnp.int32) + lanes
    plsc.store_scatter(cand_v, [pad_pos], jnp.full((L,), NEG, jnp.float32))
    plsc.store_scatter(cand_i, [pad_pos], jnp.full((L,), IBIG, jnp.int32))
    nch = (c + (L - 1)) // L
    lane0 = lanes == 0

    for s in range(K):
        def scan(j, carry):
            bv, bi, bp = carry
            v = cand_v[pl.ds(j * L, L)]
            ii = cand_i[pl.ds(j * L, L)]
            pp = lanes + j * L
            take = (v > bv) | ((v == bv) & (ii < bi))
            return (jnp.where(take, v, bv),
                    jnp.where(take, ii, bi),
                    jnp.where(take, pp, bp))

        bv, bi, bp = lax.fori_loop(
            0, nch, scan,
            (jnp.full((L,), NEG, jnp.float32),
             jnp.full((L,), IBIG, jnp.int32),
             jnp.full((L,), IBIG, jnp.int32)))
        mval = jnp.max(bv)
        eq = bv == mval
        mi = jnp.min(jnp.where(eq, bi, IBIG))
        pos = jnp.min(jnp.where(eq & (bi == mi), bp, IBIG))
        plsc.store_scatter(topv_v, [jnp.full((L,), s, jnp.int32)],
                           jnp.full((L,), mval, jnp.float32), mask=lane0)
        plsc.store_scatter(topi_v, [jnp.full((L,), s, jnp.int32)],
                           jnp.full((L,), mi, jnp.int32), mask=lane0)
        plsc.store_scatter(cand_v, [jnp.full((L,), pos, jnp.int32)],
                           jnp.full((L,), NEG, jnp.float32), mask=lane0)


def _merge_gather_mean(x2_hbm, out_row, mc_v, mc_i, idx_v, rows_v, out_v,
                       sem, nmc):
    lanes = lax.iota(jnp.int32, L)
    lane0 = lanes == 0
    for s in range(K):
        bv = jnp.full((L,), NEG, jnp.float32)
        bi = jnp.full((L,), IBIG, jnp.int32)
        bp = jnp.full((L,), IBIG, jnp.int32)
        for j in range(nmc):
            v = mc_v[pl.ds(j * L, L)]
            ii = mc_i[pl.ds(j * L, L)]
            pp = lanes + j * L
            take = (v > bv) | ((v == bv) & (ii < bi))
            bv = jnp.where(take, v, bv)
            bi = jnp.where(take, ii, bi)
            bp = jnp.where(take, pp, bp)
        mval = jnp.max(bv)
        eq = bv == mval
        mi = jnp.min(jnp.where(eq, bi, IBIG))
        pos = jnp.min(jnp.where(eq & (bi == mi), bp, IBIG))
        plsc.store_scatter(idx_v, [jnp.full((L,), s, jnp.int32)],
                           jnp.full((L,), mi, jnp.int32), mask=lane0)
        plsc.store_scatter(mc_v, [jnp.full((L,), pos, jnp.int32)],
                           jnp.full((L,), NEG, jnp.float32), mask=lane0)
    pltpu.async_copy(x2_hbm.at[idx_v], rows_v, sem).wait()
    for cc in range(C // L):
        acc = jnp.zeros((L,), jnp.float32)
        for r in range(K):
            acc = acc + rows_v[r, pl.ds(cc * L, L)]
        out_v[pl.ds(cc * L, L)] = acc * jnp.float32(1.0 / K)
    pltpu.sync_copy(out_v, out_row)


def _sca1_body(x2_hbm, cval_hbm, cidx_hbm,
               buf0, buf1, scores_v, cand_v, cand_i, topv_v, topi_v,
               dsem0, dsem1):
    cid = lax.axis_index("c")
    sid = lax.axis_index("s")
    w = sid * NC + cid
    bloc = w // WPB
    q = w % WPB
    b = TB + bloc
    base_row = b * N + q * WROWS
    lanes = lax.iota(jnp.int32, L)

    pltpu.make_async_copy(
        x2_hbm.at[pl.ds(base_row, RB)], buf0, dsem0).start()

    def chunk(o, buf, sem, nbuf, nsem):
        @pl.when(o < NCH - 1)
        def _():
            pltpu.make_async_copy(
                x2_hbm.at[pl.ds(base_row + (o + 1) * RB, RB)],
                nbuf, nsem).start()
        pltpu.make_async_copy(
            x2_hbm.at[pl.ds(base_row + o * RB, RB)], buf, sem).wait()

        lane15 = lanes == (L - 1)

        def grp(g, _):
            for r in range(L):
                row = g * L + r
                v = buf[row, pl.ds(0, L)]
                for c in range(1, C // L):
                    v = jnp.maximum(v, buf[row, pl.ds(c * L, L)])
                cm = plsc.cummax(v)
                plsc.store_scatter(
                    scores_v,
                    [jnp.full((L,), o * RB + g * L + r, jnp.int32)],
                    cm, mask=lane15)
            return 0

        lax.fori_loop(0, GRP, grp, 0)

    def step(o, _):
        @pl.when(o % 2 == 0)
        def _():
            chunk(o, buf0, dsem0, buf1, dsem1)

        @pl.when(o % 2 == 1)
        def _():
            chunk(o, buf1, dsem1, buf0, dsem0)
        return 0

    lax.fori_loop(0, NCH, step, 0)

    def p1(j, m):
        return jnp.maximum(m, scores_v[pl.ds(j * L, L)])

    m = lax.fori_loop(0, WROWS // L, p1, jnp.full((L,), NEG, jnp.float32))
    _select_top16(scores_v, cand_v, cand_i, m, base_row,
                  topv_v, topi_v, WROWS // L)
    off = w * K
    pltpu.sync_copy(topv_v, cval_hbm.at[pl.ds(off, K)])
    pltpu.sync_copy(topi_v, cidx_hbm.at[pl.ds(off, K)])


def _sca2_body(cval_hbm, cidx_hbm, x2_hbm, out_hbm,
               mc_v, mc_i, idx_v, rows_v, out_v, gsem):
    cid = lax.axis_index("c")
    sid = lax.axis_index("s")
    w = sid * NC + cid
    b2 = w % SB
    pltpu.sync_copy(cval_hbm.at[pl.ds(b2 * (WPB * K), WPB * K)], mc_v)
    pltpu.sync_copy(cidx_hbm.at[pl.ds(b2 * (WPB * K), WPB * K)], mc_i)
    _merge_gather_mean(x2_hbm, out_hbm.at[b2], mc_v, mc_i,
                       idx_v, rows_v, out_v, gsem, WPB)


def _scb_body(s1_hbm, x2_hbm, out_hbm,
              scores_v, cand_v, cand_i, topv_v, topi_v,
              mc_v, mc_i, idx_v, rows_v, out_v, gsem):
    cid = lax.axis_index("c")
    sid = lax.axis_index("s")
    w = sid * NC + cid
    b = w % TB if TB >= 32 else jnp.minimum(w, TB - 1)
    base_row = b * N

    @pl.when(w < TB)
    def _():
        def do_half(h):
            pltpu.sync_copy(s1_hbm.at[pl.ds(base_row + h * HN, HN)], scores_v)

            def p1(j, m):
                return jnp.maximum(m, scores_v[pl.ds(j * L, L)])

            m = lax.fori_loop(0, CW, p1, jnp.full((L,), NEG, jnp.float32))
            _select_top16(scores_v, cand_v, cand_i, m, base_row + h * HN,
                          topv_v, topi_v, CW)
            mc_v[pl.ds(h * L, L)] = topv_v[...]
            mc_i[pl.ds(h * L, L)] = topi_v[...]

        do_half(0)
        do_half(1)
        _merge_gather_mean(x2_hbm, out_hbm.at[b], mc_v, mc_i,
                           idx_v, rows_v, out_v, gsem, 2)


def _sel_scratch(nrows):
    return [
        pltpu.VMEM((nrows,), jnp.float32),
        pltpu.VMEM((nrows + L,), jnp.float32),
        pltpu.VMEM((nrows + L,), jnp.int32),
        pltpu.VMEM((K,), jnp.float32),
        pltpu.VMEM((K,), jnp.int32),
    ]


def _merge_scratch(nmc):
    return [
        pltpu.VMEM((nmc * K,), jnp.float32),
        pltpu.VMEM((nmc * K,), jnp.int32),
        pltpu.VMEM((K,), jnp.int32),
        pltpu.VMEM((K, C), jnp.float32),
        pltpu.VMEM((C,), jnp.float32),
        pltpu.SemaphoreType.DMA,
    ]

_MESH = dict(
    mesh=plsc.VectorSubcoreMesh(core_axis_name="c", subcore_axis_name="s"),
    compiler_params=pltpu.CompilerParams(needs_layout_passes=False),
)

_sc_a1 = functools.partial(
    pl.kernel,
    out_type=[jax.ShapeDtypeStruct((SB * WPB * K,), jnp.float32),
              jax.ShapeDtypeStruct((SB * WPB * K,), jnp.int32)],
    scratch_types=([
        pltpu.VMEM((RB, C), jnp.float32),
        pltpu.VMEM((RB, C), jnp.float32),
    ] + _sel_scratch(WROWS) + [
        pltpu.SemaphoreType.DMA,
        pltpu.SemaphoreType.DMA,
    ]),
    **_MESH,
)(_sca1_body)

_sc_a2 = functools.partial(
    pl.kernel,
    out_type=jax.ShapeDtypeStruct((SB, C), jnp.float32),
    scratch_types=_merge_scratch(WPB),
    **_MESH,
)(_sca2_body)

_sc_b = functools.partial(
    pl.kernel,
    out_type=jax.ShapeDtypeStruct((TB, C), jnp.float32),
    scratch_types=(_sel_scratch(HN) + _merge_scratch(2)),
    **_MESH,
)(_scb_body)


@jax.jit
def kernel(input):
    x2 = input.reshape(B * N, C)
    cval, cidx = _sc_a1(x2)
    out_a = _sc_a2(cval, cidx, x2)
    scores = _tc_scores_half(input)
    out_b = _sc_b(scores.reshape(TB * N), x2)
    return jnp.concatenate([out_b, out_a], axis=0)[:, None, :]


# speedup vs baseline: 2.9477x; 1.0243x over previous
import functools

import jax
import jax.numpy as jnp
from jax import lax
from jax.experimental import pallas as pl
from jax.experimental.pallas import tpu as pltpu
from jax.experimental.pallas import tpu_sc as plsc

B, N, C = 32, 8192, 128
K = 16
L = 16
NC = 2
TB = 24
SB = B - TB
WPB = 4
WROWS = N // WPB
HN = N // 2
RB = 256
NCH = WROWS // RB
GRP = RB // L
CW = HN // L

NEG = float("-inf")
IBIG = 2**31 - 1


def _scores_body(x_ref, o_ref):
    o_ref[...] = jnp.max(x_ref[...], axis=2)


def _tc_scores_half(x):
    return pl.pallas_call(
        _scores_body,
        grid=(TB // 8, 8),
        in_specs=[pl.BlockSpec((8, 1024, 128), lambda i, j: (i, j, 0))],
        out_specs=pl.BlockSpec((8, 1024), lambda i, j: (i, j)),
        out_shape=jax.ShapeDtypeStruct((TB, N), jnp.float32),
    )(x)


def _select_top16(scores_v, cand_v, cand_i, m, base, topv_v, topi_v, cw):
    lanes = lax.iota(jnp.int32, L)
    t0 = jnp.min(m)

    def p2(j, off):
        v = scores_v[pl.ds(j * L, L)]
        msk = v >= t0
        pos = off + plsc.cumsum(msk.astype(jnp.int32)) - 1
        plsc.store_scatter(cand_v, [pos], v, mask=msk)
        plsc.store_scatter(cand_i, [pos], base + j * L + lanes, mask=msk)
        return off + jnp.max(plsc.all_reduce_population_count(msk))

    c = lax.fori_loop(0, cw, p2, jnp.int32(0))

    pad_pos = jnp.full((L,), c, jnp.int32) + lanes
    plsc.store_scatter(cand_v, [pad_pos], jnp.full((L,), NEG, jnp.float32))
    plsc.store_scatter(cand_i, [pad_pos], jnp.full((L,), IBIG, jnp.int32))
    nch = (c + (L - 1)) // L
    lane0 = lanes == 0

    for s in range(K):
        def scan(j, carry):
            bv, bi, bp = carry
            v = cand_v[pl.ds(j * L, L)]
            ii = cand_i[pl.ds(j * L, L)]
            pp = lanes + j * L
            take = (v > bv) | ((v == bv) & (ii < bi))
            return (jnp.where(take, v, bv),
                    jnp.where(take, ii, bi),
                    jnp.where(take, pp, bp))

        bv, bi, bp = lax.fori_loop(
            0, nch, scan,
            (jnp.full((L,), NEG, jnp.float32),
             jnp.full((L,), IBIG, jnp.int32),
             jnp.full((L,), IBIG, jnp.int32)))
        mval = jnp.max(bv)
        eq = bv == mval
        mi = jnp.min(jnp.where(eq, bi, IBIG))
        pos = jnp.min(jnp.where(eq & (bi == mi), bp, IBIG))
        plsc.store_scatter(topv_v, [jnp.full((L,), s, jnp.int32)],
                           jnp.full((L,), mval, jnp.float32), mask=lane0)
        plsc.store_scatter(topi_v, [jnp.full((L,), s, jnp.int32)],
                           jnp.full((L,), mi, jnp.int32), mask=lane0)
        plsc.store_scatter(cand_v, [jnp.full((L,), pos, jnp.int32)],
                           jnp.full((L,), NEG, jnp.float32), mask=lane0)


def _merge_gather_mean(x2_hbm, out_row, mc_v, mc_i, idx_v, rows_v, out_v,
                       sem, nmc):
    lanes = lax.iota(jnp.int32, L)
    lane0 = lanes == 0
    for s in range(K):
        bv = jnp.full((L,), NEG, jnp.float32)
        bi = jnp.full((L,), IBIG, jnp.int32)
        bp = jnp.full((L,), IBIG, jnp.int32)
        for j in range(nmc):
            v = mc_v[pl.ds(j * L, L)]
            ii = mc_i[pl.ds(j * L, L)]
            pp = lanes + j * L
            take = (v > bv) | ((v == bv) & (ii < bi))
            bv = jnp.where(take, v, bv)
            bi = jnp.where(take, ii, bi)
            bp = jnp.where(take, pp, bp)
        mval = jnp.max(bv)
        eq = bv == mval
        mi = jnp.min(jnp.where(eq, bi, IBIG))
        pos = jnp.min(jnp.where(eq & (bi == mi), bp, IBIG))
        plsc.store_scatter(idx_v, [jnp.full((L,), s, jnp.int32)],
                           jnp.full((L,), mi, jnp.int32), mask=lane0)
        plsc.store_scatter(mc_v, [jnp.full((L,), pos, jnp.int32)],
                           jnp.full((L,), NEG, jnp.float32), mask=lane0)
    pltpu.async_copy(x2_hbm.at[idx_v], rows_v, sem).wait()
    for cc in range(C // L):
        acc = jnp.zeros((L,), jnp.float32)
        for r in range(K):
            acc = acc + rows_v[r, pl.ds(cc * L, L)]
        out_v[pl.ds(cc * L, L)] = acc * jnp.float32(1.0 / K)
    pltpu.sync_copy(out_v, out_row)


def _sca1_body(x2_hbm, cval_hbm, cidx_hbm,
               buf0, buf1, scores_v, cand_v, cand_i, topv_v, topi_v,
               dsem0, dsem1):
    cid = lax.axis_index("c")
    sid = lax.axis_index("s")
    w = sid * NC + cid
    bloc = w // WPB
    q = w % WPB
    b = TB + bloc
    base_row = b * N + q * WROWS
    lanes = lax.iota(jnp.int32, L)

    pltpu.make_async_copy(
        x2_hbm.at[pl.ds(base_row, RB)], buf0, dsem0).start()

    def chunk(o, buf, sem, nbuf, nsem):
        @pl.when(o < NCH - 1)
        def _():
            pltpu.make_async_copy(
                x2_hbm.at[pl.ds(base_row + (o + 1) * RB, RB)],
                nbuf, nsem).start()
        pltpu.make_async_copy(
            x2_hbm.at[pl.ds(base_row + o * RB, RB)], buf, sem).wait()

        lane15 = lanes == (L - 1)

        def grp(g, _):
            for r in range(L):
                row = g * L + r
                v = buf[row, pl.ds(0, L)]
                for c in range(1, C // L):
                    v = jnp.maximum(v, buf[row, pl.ds(c * L, L)])
                cm = plsc.cummax(v)
                plsc.store_scatter(
                    scores_v,
                    [jnp.full((L,), o * RB + g * L + r, jnp.int32)],
                    cm, mask=lane15)
            return 0

        lax.fori_loop(0, GRP, grp, 0)

    def step(o, _):
        @pl.when(o % 2 == 0)
        def _():
            chunk(o, buf0, dsem0, buf1, dsem1)

        @pl.when(o % 2 == 1)
        def _():
            chunk(o, buf1, dsem1, buf0, dsem0)
        return 0

    lax.fori_loop(0, NCH, step, 0)

    def p1(j, m):
        return jnp.maximum(m, scores_v[pl.ds(j * L, L)])

    m = lax.fori_loop(0, WROWS // L, p1, jnp.full((L,), NEG, jnp.float32))
    _select_top16(scores_v, cand_v, cand_i, m, base_row,
                  topv_v, topi_v, WROWS // L)
    off = w * K
    pltpu.sync_copy(topv_v, cval_hbm.at[pl.ds(off, K)])
    pltpu.sync_copy(topi_v, cidx_hbm.at[pl.ds(off, K)])


def _scb1_body(s1_hbm, cval_hbm, cidx_hbm,
               scores_v, cand_v, cand_i, topv_v, topi_v):
    cid = lax.axis_index("c")
    sid = lax.axis_index("s")
    w = sid * NC + cid
    for jj in range(TB * WPB // 32):
        job = w + jj * 32
        b = job // WPB
        q = job % WPB
        base_row = b * N + q * WROWS
        pltpu.sync_copy(s1_hbm.at[pl.ds(base_row, WROWS)], scores_v)

        def p1(j, m):
            return jnp.maximum(m, scores_v[pl.ds(j * L, L)])

        m = lax.fori_loop(0, WROWS // L, p1, jnp.full((L,), NEG, jnp.float32))
        _select_top16(scores_v, cand_v, cand_i, m, base_row,
                      topv_v, topi_v, WROWS // L)
        pltpu.sync_copy(topv_v, cval_hbm.at[pl.ds(job * K, K)])
        pltpu.sync_copy(topi_v, cidx_hbm.at[pl.ds(job * K, K)])


def _scm_body(cvA_hbm, ciA_hbm, cvB_hbm, ciB_hbm, x2_hbm, out_hbm,
              mc_v, mc_i, idx_v, rows_v, out_v, gsem):
    cid = lax.axis_index("c")
    sid = lax.axis_index("s")
    w = sid * NC + cid

    @pl.when(w < TB)
    def _():
        off = w * (WPB * K)
        pltpu.sync_copy(cvB_hbm.at[pl.ds(off, WPB * K)], mc_v)
        pltpu.sync_copy(ciB_hbm.at[pl.ds(off, WPB * K)], mc_i)

    @pl.when(w >= TB)
    def _():
        off = (w - TB) * (WPB * K)
        pltpu.sync_copy(cvA_hbm.at[pl.ds(off, WPB * K)], mc_v)
        pltpu.sync_copy(ciA_hbm.at[pl.ds(off, WPB * K)], mc_i)

    _merge_gather_mean(x2_hbm, out_hbm.at[w], mc_v, mc_i,
                       idx_v, rows_v, out_v, gsem, WPB)


def _sel_scratch(nrows):
    return [
        pltpu.VMEM((nrows,), jnp.float32),
        pltpu.VMEM((nrows + L,), jnp.float32),
        pltpu.VMEM((nrows + L,), jnp.int32),
        pltpu.VMEM((K,), jnp.float32),
        pltpu.VMEM((K,), jnp.int32),
    ]


def _merge_scratch(nmc):
    return [
        pltpu.VMEM((nmc * K,), jnp.float32),
        pltpu.VMEM((nmc * K,), jnp.int32),
        pltpu.VMEM((K,), jnp.int32),
        pltpu.VMEM((K, C), jnp.float32),
        pltpu.VMEM((C,), jnp.float32),
        pltpu.SemaphoreType.DMA,
    ]

_MESH = dict(
    mesh=plsc.VectorSubcoreMesh(core_axis_name="c", subcore_axis_name="s"),
    compiler_params=pltpu.CompilerParams(needs_layout_passes=False),
)

_sc_a1 = functools.partial(
    pl.kernel,
    out_type=[jax.ShapeDtypeStruct((SB * WPB * K,), jnp.float32),
              jax.ShapeDtypeStruct((SB * WPB * K,), jnp.int32)],
    scratch_types=([
        pltpu.VMEM((RB, C), jnp.float32),
        pltpu.VMEM((RB, C), jnp.float32),
    ] + _sel_scratch(WROWS) + [
        pltpu.SemaphoreType.DMA,
        pltpu.SemaphoreType.DMA,
    ]),
    **_MESH,
)(_sca1_body)

_sc_b1 = functools.partial(
    pl.kernel,
    out_type=[jax.ShapeDtypeStruct((TB * WPB * K,), jnp.float32),
              jax.ShapeDtypeStruct((TB * WPB * K,), jnp.int32)],
    scratch_types=_sel_scratch(WROWS),
    **_MESH,
)(_scb1_body)

_sc_merge = functools.partial(
    pl.kernel,
    out_type=jax.ShapeDtypeStruct((B, C), jnp.float32),
    scratch_types=_merge_scratch(WPB),
    **_MESH,
)(_scm_body)


@jax.jit
def kernel(input):
    x2 = input.reshape(B * N, C)
    cvA, ciA = _sc_a1(x2)
    scores = _tc_scores_half(input)
    cvB, ciB = _sc_b1(scores.reshape(TB * N))
    out = _sc_merge(cvA, ciA, cvB, ciB, x2)
    return out[:, None, :]
